# baseline (speedup 1.0000x reference)
import functools

import numpy as np
import jax
import jax.numpy as jnp
from jax import lax
from jax.experimental import pallas as pl
from jax.experimental.pallas import tpu as pltpu


def _fused_kernel(x_ref, selb_ref, wb1_ref, b1_ref, selb2_ref, wb2_ref,
                  b2_ref, wfc_ref, bfc_ref, o_ref,
                  xcomp, xwin, s1rhs, acc1, h1s, h1cs, hwin2, s2rhs,
                  acc2, h2s):
    xcomp[0:1024, :] = jnp.dot(x_ref[0], selb_ref[...],
                               preferred_element_type=jnp.float32
                               ).astype(jnp.bfloat16)
    xcomp[1024:1056, :] = jnp.zeros((32, 2048), jnp.bfloat16)
    h2s[39:48, :, :] = jnp.zeros((9, 10, 39), jnp.float32)

    def _s1_rows(t, g_count):
        for g in range(g_count):
            rhs = s1rhs.at[g % 2]
            acc = acc1.at[g % 2]
            for q in range(4):
                for b in range(8):
                    r0 = 64 * q + 8 * b
                    rhs[r0:r0 + 8, :] = xwin[5 * (4 * g + q):
                                             5 * (4 * g + q) + 8,
                                             256 * b:256 * (b + 1)]
            acc[...] = jnp.dot(wb1_ref[...], rhs[...],
                               preferred_element_type=jnp.float32)
            for q in range(4):
                m = acc[200 * q:200 * q + 8, :]
                for p in range(1, 25):
                    m = jnp.maximum(m, acc[200 * q + 8 * p:
                                           200 * q + 8 * p + 8, :])
                h = jnp.maximum(m + b1_ref[...], 0.0).astype(jnp.bfloat16)
                s = 4 * g + q
                for co in range(5):
                    h1s[t, 16 * co + s:16 * co + s + 1, :] = \
                        h[co:co + 1, 0:204]

    def body1(t, carry):
        xwin[...] = xcomp[pl.ds(80 * t, 96), :]
        _s1_rows(t, 4)
        return carry

    lax.fori_loop(0, 12, body1, 0)
    xwin[...] = xcomp[960:1056, :]
    _s1_rows(12, 3)

    for t in range(13):
        hc = jnp.dot(h1s[t], selb2_ref[...],
                     preferred_element_type=jnp.float32).astype(jnp.bfloat16)
        for cin in range(5):
            h1cs[cin, 16 * t:16 * t + 16, :] = hc[16 * cin:16 * cin + 16, :]

    def _s2_rows(t2, s2_count):
        for s2 in range(s2_count):
            rhs = s2rhs.at[s2 % 2]
            acc = acc2.at[s2 % 2]
            for b2 in range(12):
                for cin in range(5):
                    r0 = 60 * b2 + 12 * cin
                    rhs[r0:r0 + 12, :] = hwin2[cin, 5 * s2:5 * s2 + 12,
                                               128 * b2:128 * (b2 + 1)]
            acc[...] = jnp.dot(wb2_ref[...], rhs[...],
                               preferred_element_type=jnp.float32)
            m = acc[0:16, :]
            for p in range(1, 25):
                m = jnp.maximum(m, acc[16 * p:16 * p + 16, :])
            h2 = jnp.maximum(m + b2_ref[...], 0.0)
            h2s[16 * t2 + s2, :, :] = h2[0:10, 0:39]

    def body2(t2, carry):
        hwin2[...] = h1cs[:, pl.ds(80 * t2, 96), :]
        _s2_rows(t2, 16)
        return carry

    lax.fori_loop(0, 2, body2, 0)
    hwin2[:, 0:48, :] = h1cs[:, 160:208, :]
    _s2_rows(2, 7)

    h2all = h2s[...]
    s0 = jnp.sum(h2all * wfc_ref[0], keepdims=True).reshape(1, 1, 1)
    s1_ = jnp.sum(h2all * wfc_ref[1], keepdims=True).reshape(1, 1, 1)
    o_ref[...] = jnp.concatenate([s0, s1_], axis=2) + bfc_ref[...]


def _expand_conv_weights(w, cpad):
    cout, cin, k, _ = w.shape
    nph = k + 4
    rows = []
    for p in range(25):
        di, dj = divmod(p, 5)
        slab = jnp.zeros((cout, cin, nph, nph), w.dtype)
        slab = slab.at[:, :, di:di + k, dj:dj + k].set(w)
        slab = slab.reshape(cout, cin * nph * nph)
        rows.append(jnp.pad(slab, ((0, cpad - cout), (0, 0))))
    return jnp.concatenate(rows, axis=0)


def _selection_matrix(src, nphase, npool, win):
    c = np.arange(src).reshape(src, 1)
    t = np.arange(nphase * win).reshape(1, nphase * win)
    b, j = t // win, t % win
    valid = j < npool
    sel = (c == b + 5 * j) & valid
    return jnp.asarray(sel, jnp.bfloat16)


@jax.jit
def _forward(label, w1, b1, w2, b2, wfc, bfc):
    N = label.shape[0]
    xbf = label[:, 0].astype(jnp.bfloat16)

    selb = _selection_matrix(1024, 8, 204, 256)
    selb2 = _selection_matrix(204, 12, 39, 128)

    wb1 = _expand_conv_weights(w1, cpad=8)
    perm1 = np.arange(64).reshape(8, 8).T.reshape(64)
    wb1p = wb1[:, perm1].astype(jnp.bfloat16)
    wb1q = jnp.zeros((800, 256), jnp.bfloat16)
    for q in range(4):
        wb1q = wb1q.at[200 * q:200 * q + 200, 64 * q:64 * q + 64].set(wb1p)
    b1c = jnp.pad(b1, (0, 3)).reshape(8, 1)

    wb2 = _expand_conv_weights(w2, cpad=16)
    perm2 = (np.arange(720).reshape(5, 12, 12)
             .transpose(2, 0, 1).reshape(720))
    wb2p = wb2[:, perm2].astype(jnp.bfloat16)
    b2c = jnp.pad(b2, (0, 6)).reshape(16, 1)

    wfc4 = wfc.reshape(2, 10, 39, 39).transpose(0, 2, 1, 3)
    wfc5 = jnp.pad(wfc4, ((0, 0), (0, 9), (0, 0), (0, 0)))
    bfcr = bfc.reshape(1, 1, 2)

    out = pl.pallas_call(
        _fused_kernel,
        out_shape=jax.ShapeDtypeStruct((N, 1, 2), jnp.float32),
        grid=(N,),
        in_specs=[
            pl.BlockSpec((1, 1024, 1024), lambda n: (n, 0, 0)),
            pl.BlockSpec((1024, 2048), lambda n: (0, 0)),
            pl.BlockSpec((800, 256), lambda n: (0, 0)),
            pl.BlockSpec((8, 1), lambda n: (0, 0)),
            pl.BlockSpec((204, 1536), lambda n: (0, 0)),
            pl.BlockSpec((400, 720), lambda n: (0, 0)),
            pl.BlockSpec((16, 1), lambda n: (0, 0)),
            pl.BlockSpec((2, 48, 10, 39), lambda n: (0, 0, 0, 0)),
            pl.BlockSpec((1, 1, 2), lambda n: (0, 0, 0)),
        ],
        out_specs=pl.BlockSpec((1, 1, 2), lambda n: (n, 0, 0)),
        scratch_shapes=[
            pltpu.VMEM((1056, 2048), jnp.bfloat16),
            pltpu.VMEM((96, 2048), jnp.bfloat16),
            pltpu.VMEM((2, 256, 256), jnp.bfloat16),
            pltpu.VMEM((2, 800, 256), jnp.float32),
            pltpu.VMEM((13, 80, 204), jnp.bfloat16),
            pltpu.VMEM((5, 208, 1536), jnp.bfloat16),
            pltpu.VMEM((5, 96, 1536), jnp.bfloat16),
            pltpu.VMEM((2, 720, 128), jnp.bfloat16),
            pltpu.VMEM((2, 400, 128), jnp.float32),
            pltpu.VMEM((48, 10, 39), jnp.float32),
        ],
        compiler_params=pltpu.CompilerParams(
            dimension_semantics=("parallel",),
            vmem_limit_bytes=64 * 1024 * 1024,
        ),
    )(xbf, selb, wb1q, b1c, selb2, wb2p, b2c, wfc5, bfcr)
    return out.reshape(N, 2)


def kernel(label, w1, b1, w2, b2, wfc, bfc):
    return _forward(label, w1, b1, w2, b2, wfc, bfc)


# speedup vs baseline: 1.0222x; 1.0222x over previous
import functools

import numpy as np
import jax
import jax.numpy as jnp
from jax import lax
from jax.experimental import pallas as pl
from jax.experimental.pallas import tpu as pltpu


def _fused_kernel(x_ref, selb_ref, wb1_ref, b1_ref, selb2_ref, wb2_ref,
                  b2_ref, wfc_ref, bfc_ref, o_ref,
                  xcomp, xwin, s1rhs, acc1, h1s, h1cs, hwin2, s2rhs,
                  acc2, h2s):
    xcomp[0:1024, :] = jnp.dot(x_ref[0], selb_ref[...],
                               preferred_element_type=jnp.float32
                               ).astype(jnp.bfloat16)
    xcomp[1024:1056, :] = jnp.zeros((32, 2048), jnp.bfloat16)
    h2s[39:48, :, :] = jnp.zeros((9, 10, 39), jnp.float32)

    def _s1_rows(t, s_count):
        for s in range(s_count):
            rhs = s1rhs.at[s % 2]
            acc = acc1.at[s % 2]
            for b in range(8):
                rhs[8 * b:8 * b + 8, :] = xwin[5 * s:5 * s + 8,
                                               256 * b:256 * (b + 1)]
            acc[...] = jnp.dot(wb1_ref[...], rhs[...],
                               preferred_element_type=jnp.float32)
            m = acc[0:8, :]
            for p in range(1, 25):
                m = jnp.maximum(m, acc[8 * p:8 * p + 8, :])
            h = jnp.maximum(m + b1_ref[...], 0.0).astype(jnp.bfloat16)
            for co in range(5):
                h1s[t, 16 * co + s:16 * co + s + 1, :] = h[co:co + 1, 0:204]

    def body1(t, carry):
        xwin[...] = xcomp[pl.ds(80 * t, 96), :]
        _s1_rows(t, 16)
        return carry

    lax.fori_loop(0, 12, body1, 0)
    xwin[...] = xcomp[960:1056, :]
    _s1_rows(12, 12)

    for t in range(13):
        hc = jnp.dot(h1s[t], selb2_ref[...],
                     preferred_element_type=jnp.float32).astype(jnp.bfloat16)
        for cin in range(5):
            h1cs[cin, 16 * t:16 * t + 16, :] = hc[16 * cin:16 * cin + 16, :]

    def _s2_rows(t2, s2_count):
        for s2 in range(s2_count):
            rhs = s2rhs.at[s2 % 2]
            acc = acc2.at[s2 % 2]
            for b2 in range(12):
                for cin in range(5):
                    r0 = 60 * b2 + 12 * cin
                    rhs[r0:r0 + 12, :] = hwin2[cin, 5 * s2:5 * s2 + 12,
                                               128 * b2:128 * (b2 + 1)]
            acc[...] = jnp.dot(wb2_ref[...], rhs[...],
                               preferred_element_type=jnp.float32)
            m = acc[0:16, :]
            for p in range(1, 25):
                m = jnp.maximum(m, acc[16 * p:16 * p + 16, :])
            h2 = jnp.maximum(m + b2_ref[...], 0.0)
            h2s[16 * t2 + s2, :, :] = h2[0:10, 0:39]

    def body2(t2, carry):
        hwin2[...] = h1cs[:, pl.ds(80 * t2, 96), :]
        _s2_rows(t2, 16)
        return carry

    lax.fori_loop(0, 2, body2, 0)
    hwin2[:, 0:48, :] = h1cs[:, 160:208, :]
    _s2_rows(2, 7)

    h2all = h2s[...]
    s0 = jnp.sum(h2all * wfc_ref[0], keepdims=True).reshape(1, 1, 1)
    s1_ = jnp.sum(h2all * wfc_ref[1], keepdims=True).reshape(1, 1, 1)
    o_ref[...] = jnp.concatenate([s0, s1_], axis=2) + bfc_ref[...]


def _expand_conv_weights(w, cpad):
    cout, cin, k, _ = w.shape
    nph = k + 4
    rows = []
    for p in range(25):
        di, dj = divmod(p, 5)
        slab = jnp.zeros((cout, cin, nph, nph), w.dtype)
        slab = slab.at[:, :, di:di + k, dj:dj + k].set(w)
        slab = slab.reshape(cout, cin * nph * nph)
        rows.append(jnp.pad(slab, ((0, cpad - cout), (0, 0))))
    return jnp.concatenate(rows, axis=0)


def _selection_matrix(src, nphase, npool, win):
    c = np.arange(src).reshape(src, 1)
    t = np.arange(nphase * win).reshape(1, nphase * win)
    b, j = t // win, t % win
    valid = j < npool
    sel = (c == b + 5 * j) & valid
    return jnp.asarray(sel, jnp.bfloat16)


@jax.jit
def _forward(label, w1, b1, w2, b2, wfc, bfc):
    N = label.shape[0]
    xbf = label[:, 0].astype(jnp.bfloat16)

    selb = _selection_matrix(1024, 8, 204, 256)
    selb2 = _selection_matrix(204, 12, 39, 128)

    wb1 = _expand_conv_weights(w1, cpad=8)
    perm1 = np.arange(64).reshape(8, 8).T.reshape(64)
    wb1p = wb1[:, perm1].astype(jnp.bfloat16)
    b1c = jnp.pad(b1, (0, 3)).reshape(8, 1)

    wb2 = _expand_conv_weights(w2, cpad=16)
    perm2 = (np.arange(720).reshape(5, 12, 12)
             .transpose(2, 0, 1).reshape(720))
    wb2p = wb2[:, perm2].astype(jnp.bfloat16)
    b2c = jnp.pad(b2, (0, 6)).reshape(16, 1)

    wfc4 = wfc.reshape(2, 10, 39, 39).transpose(0, 2, 1, 3)
    wfc5 = jnp.pad(wfc4, ((0, 0), (0, 9), (0, 0), (0, 0)))
    bfcr = bfc.reshape(1, 1, 2)

    out = pl.pallas_call(
        _fused_kernel,
        out_shape=jax.ShapeDtypeStruct((N, 1, 2), jnp.float32),
        grid=(N,),
        in_specs=[
            pl.BlockSpec((1, 1024, 1024), lambda n: (n, 0, 0)),
            pl.BlockSpec((1024, 2048), lambda n: (0, 0)),
            pl.BlockSpec((200, 64), lambda n: (0, 0)),
            pl.BlockSpec((8, 1), lambda n: (0, 0)),
            pl.BlockSpec((204, 1536), lambda n: (0, 0)),
            pl.BlockSpec((400, 720), lambda n: (0, 0)),
            pl.BlockSpec((16, 1), lambda n: (0, 0)),
            pl.BlockSpec((2, 48, 10, 39), lambda n: (0, 0, 0, 0)),
            pl.BlockSpec((1, 1, 2), lambda n: (0, 0, 0)),
        ],
        out_specs=pl.BlockSpec((1, 1, 2), lambda n: (n, 0, 0)),
        scratch_shapes=[
            pltpu.VMEM((1056, 2048), jnp.bfloat16),
            pltpu.VMEM((96, 2048), jnp.bfloat16),
            pltpu.VMEM((2, 64, 256), jnp.bfloat16),
            pltpu.VMEM((2, 200, 256), jnp.float32),
            pltpu.VMEM((13, 80, 204), jnp.bfloat16),
            pltpu.VMEM((5, 208, 1536), jnp.bfloat16),
            pltpu.VMEM((5, 96, 1536), jnp.bfloat16),
            pltpu.VMEM((2, 720, 128), jnp.bfloat16),
            pltpu.VMEM((2, 400, 128), jnp.float32),
            pltpu.VMEM((48, 10, 39), jnp.float32),
        ],
        compiler_params=pltpu.CompilerParams(
            dimension_semantics=("parallel",),
            vmem_limit_bytes=64 * 1024 * 1024,
        ),
    )(xbf, selb, wb1p, b1c, selb2, wb2p, b2c, wfc5, bfcr)
    return out.reshape(N, 2)


def kernel(label, w1, b1, w2, b2, wfc, bfc):
    return _forward(label, w1, b1, w2, b2, wfc, bfc)


# speedup vs baseline: 1.0398x; 1.0172x over previous
import functools

import numpy as np
import jax
import jax.numpy as jnp
from jax import lax
from jax.experimental import pallas as pl
from jax.experimental.pallas import tpu as pltpu


def _fused_kernel(x_ref, selb_ref, wb1_ref, b1_ref, selb2_ref, wb2_ref,
                  b2_ref, wfc_ref, bfc_ref, o_ref,
                  xcomp, xwin, s1rhs, acc1, h1s, h1cs, hwin2, s2rhs,
                  acc2, h2s):
    xcomp[0:1024, :] = jnp.dot(x_ref[0], selb_ref[...],
                               preferred_element_type=jnp.float32
                               ).astype(jnp.bfloat16)
    xcomp[1024:1056, :] = jnp.zeros((32, 2048), jnp.bfloat16)
    h2s[39:48, :, :] = jnp.zeros((9, 10, 39), jnp.float32)

    def _s1_rows(t, s_count):
        for s in range(s_count):
            rhs = s1rhs
            acc = acc1
            for b in range(8):
                rhs[8 * b:8 * b + 8, :] = xwin[5 * s:5 * s + 8,
                                               256 * b:256 * (b + 1)]
            acc[...] = jnp.dot(wb1_ref[...], rhs[...],
                               preferred_element_type=jnp.float32)
            m = acc[0:8, :]
            for p in range(1, 25):
                m = jnp.maximum(m, acc[8 * p:8 * p + 8, :])
            h = jnp.maximum(m + b1_ref[...], 0.0).astype(jnp.bfloat16)
            for co in range(5):
                h1s[t, 16 * co + s:16 * co + s + 1, :] = h[co:co + 1, 0:204]

    def body1(t, carry):
        xwin[...] = xcomp[pl.ds(80 * t, 96), :]
        _s1_rows(t, 16)
        return carry

    lax.fori_loop(0, 12, body1, 0)
    xwin[...] = xcomp[960:1056, :]
    _s1_rows(12, 12)

    for t in range(13):
        hc = jnp.dot(h1s[t], selb2_ref[...],
                     preferred_element_type=jnp.float32).astype(jnp.bfloat16)
        for cin in range(5):
            h1cs[cin, 16 * t:16 * t + 16, :] = hc[16 * cin:16 * cin + 16, :]

    def _s2_rows(t2, s2_count):
        for s2 in range(s2_count):
            rhs = s2rhs
            acc = acc2
            for b2 in range(12):
                for cin in range(5):
                    r0 = 60 * b2 + 12 * cin
                    rhs[r0:r0 + 12, :] = hwin2[cin, 5 * s2:5 * s2 + 12,
                                               128 * b2:128 * (b2 + 1)]
            acc[...] = jnp.dot(wb2_ref[...], rhs[...],
                               preferred_element_type=jnp.float32)
            m = acc[0:16, :]
            for p in range(1, 25):
                m = jnp.maximum(m, acc[16 * p:16 * p + 16, :])
            h2 = jnp.maximum(m + b2_ref[...], 0.0)
            h2s[16 * t2 + s2, :, :] = h2[0:10, 0:39]

    def body2(t2, carry):
        hwin2[...] = h1cs[:, pl.ds(80 * t2, 96), :]
        _s2_rows(t2, 16)
        return carry

    lax.fori_loop(0, 2, body2, 0)
    hwin2[:, 0:48, :] = h1cs[:, 160:208, :]
    _s2_rows(2, 7)

    h2all = h2s[...]
    s0 = jnp.sum(h2all * wfc_ref[0], keepdims=True).reshape(1, 1, 1)
    s1_ = jnp.sum(h2all * wfc_ref[1], keepdims=True).reshape(1, 1, 1)
    o_ref[...] = jnp.concatenate([s0, s1_], axis=2) + bfc_ref[...]


def _expand_conv_weights(w, cpad):
    cout, cin, k, _ = w.shape
    nph = k + 4
    rows = []
    for p in range(25):
        di, dj = divmod(p, 5)
        slab = jnp.zeros((cout, cin, nph, nph), w.dtype)
        slab = slab.at[:, :, di:di + k, dj:dj + k].set(w)
        slab = slab.reshape(cout, cin * nph * nph)
        rows.append(jnp.pad(slab, ((0, cpad - cout), (0, 0))))
    return jnp.concatenate(rows, axis=0)


def _selection_matrix(src, nphase, npool, win):
    c = np.arange(src).reshape(src, 1)
    t = np.arange(nphase * win).reshape(1, nphase * win)
    b, j = t // win, t % win
    valid = j < npool
    sel = (c == b + 5 * j) & valid
    return jnp.asarray(sel, jnp.bfloat16)


@jax.jit
def _forward(label, w1, b1, w2, b2, wfc, bfc):
    N = label.shape[0]
    xbf = label[:, 0].astype(jnp.bfloat16)

    selb = _selection_matrix(1024, 8, 204, 256)
    selb2 = _selection_matrix(204, 12, 39, 128)

    wb1 = _expand_conv_weights(w1, cpad=8)
    perm1 = np.arange(64).reshape(8, 8).T.reshape(64)
    wb1p = wb1[:, perm1].astype(jnp.bfloat16)
    b1c = jnp.pad(b1, (0, 3)).reshape(8, 1)

    wb2 = _expand_conv_weights(w2, cpad=16)
    perm2 = (np.arange(720).reshape(5, 12, 12)
             .transpose(2, 0, 1).reshape(720))
    wb2p = wb2[:, perm2].astype(jnp.bfloat16)
    b2c = jnp.pad(b2, (0, 6)).reshape(16, 1)

    wfc4 = wfc.reshape(2, 10, 39, 39).transpose(0, 2, 1, 3)
    wfc5 = jnp.pad(wfc4, ((0, 0), (0, 9), (0, 0), (0, 0)))
    bfcr = bfc.reshape(1, 1, 2)

    out = pl.pallas_call(
        _fused_kernel,
        out_shape=jax.ShapeDtypeStruct((N, 1, 2), jnp.float32),
        grid=(N,),
        in_specs=[
            pl.BlockSpec((1, 1024, 1024), lambda n: (n, 0, 0)),
            pl.BlockSpec((1024, 2048), lambda n: (0, 0)),
            pl.BlockSpec((200, 64), lambda n: (0, 0)),
            pl.BlockSpec((8, 1), lambda n: (0, 0)),
            pl.BlockSpec((204, 1536), lambda n: (0, 0)),
            pl.BlockSpec((400, 720), lambda n: (0, 0)),
            pl.BlockSpec((16, 1), lambda n: (0, 0)),
            pl.BlockSpec((2, 48, 10, 39), lambda n: (0, 0, 0, 0)),
            pl.BlockSpec((1, 1, 2), lambda n: (0, 0, 0)),
        ],
        out_specs=pl.BlockSpec((1, 1, 2), lambda n: (n, 0, 0)),
        scratch_shapes=[
            pltpu.VMEM((1056, 2048), jnp.bfloat16),
            pltpu.VMEM((96, 2048), jnp.bfloat16),
            pltpu.VMEM((64, 256), jnp.bfloat16),
            pltpu.VMEM((200, 256), jnp.float32),
            pltpu.VMEM((13, 80, 204), jnp.bfloat16),
            pltpu.VMEM((5, 208, 1536), jnp.bfloat16),
            pltpu.VMEM((5, 96, 1536), jnp.bfloat16),
            pltpu.VMEM((720, 128), jnp.bfloat16),
            pltpu.VMEM((400, 128), jnp.float32),
            pltpu.VMEM((48, 10, 39), jnp.float32),
        ],
        compiler_params=pltpu.CompilerParams(
            dimension_semantics=("parallel",),
            vmem_limit_bytes=64 * 1024 * 1024,
        ),
    )(xbf, selb, wb1p, b1c, selb2, wb2p, b2c, wfc5, bfcr)
    return out.reshape(N, 2)


def kernel(label, w1, b1, w2, b2, wfc, bfc):
    return _forward(label, w1, b1, w2, b2, wfc, bfc)
